# baseline (speedup 1.0000x reference)
import functools
import math

import jax
import jax.numpy as jnp
from jax import lax
from jax.experimental import pallas as pl
from jax.experimental.pallas import tpu as pltpu
from jax.experimental.pallas import tpu_sc as plsc

DIM = 64
SCALE = math.sqrt(DIM)

NC = 2
NS = 16
NW = NC * NS
CHUNK = 128
LANES = 16


@functools.lru_cache(maxsize=None)
def _build(n_tokens: int, vocab: int):
    per_w = n_tokens // NW
    n_chunks = per_w // CHUNK
    assert per_w * NW == n_tokens and n_chunks * CHUNK == per_w

    mesh = plsc.VectorSubcoreMesh(core_axis_name="c", subcore_axis_name="s")

    @functools.partial(
        pl.kernel,
        mesh=mesh,
        compiler_params=pltpu.CompilerParams(use_tc_tiling_on_sc=False),
        out_type=jax.ShapeDtypeStruct((n_tokens, DIM), jnp.float32),
        scratch_types=[
            pltpu.VMEM((n_chunks, CHUNK), jnp.int32),
            pltpu.VMEM((CHUNK, DIM), jnp.float32),
            pltpu.SemaphoreType.DMA,
        ],
    )
    def emb_kernel(idx_hbm, table_hbm, out_hbm, idx_v, rows_v, sem):
        wid = lax.axis_index("s") * NC + lax.axis_index("c")
        base = wid * per_w
        pltpu.sync_copy(idx_hbm.at[wid], idx_v)

        def chunk_body(j, carry):
            pltpu.async_copy(table_hbm.at[idx_v.at[j]], rows_v, sem).wait()

            def row_body(r, c2):
                for c in range(DIM // LANES):
                    sl = pl.ds(c * LANES, LANES)
                    rows_v[r, sl] = rows_v[r, sl] * SCALE
                return c2

            lax.fori_loop(0, CHUNK, row_body, 0, unroll=2)
            pltpu.sync_copy(rows_v, out_hbm.at[pl.ds(base + j * CHUNK, CHUNK)])
            return carry

        lax.fori_loop(0, n_chunks, chunk_body, 0)

    return emb_kernel


def kernel(tokens, W):
    n_tokens = tokens.shape[0] * tokens.shape[1]
    idx = tokens.reshape(NW, n_tokens // (NW * CHUNK), CHUNK).astype(jnp.int32)
    out = _build(n_tokens, W.shape[0])(idx, W)
    return out.reshape(tokens.shape[0], tokens.shape[1], DIM)


# speedup vs baseline: 1.1655x; 1.1655x over previous
import functools
import math

import jax
import jax.numpy as jnp
from jax import lax
from jax.experimental import pallas as pl
from jax.experimental.pallas import tpu as pltpu
from jax.experimental.pallas import tpu_sc as plsc

DIM = 64
SCALE = math.sqrt(DIM)

NC = 2
NS = 16
NW = NC * NS
CHUNK = 128
LANES = 16
NB = 4


@functools.lru_cache(maxsize=None)
def _build(n_tokens: int, vocab: int):
    per_w = n_tokens // NW
    n_chunks = per_w // CHUNK
    assert per_w * NW == n_tokens and n_chunks * CHUNK == per_w
    assert n_chunks % NB == 0

    mesh = plsc.VectorSubcoreMesh(core_axis_name="c", subcore_axis_name="s")

    scratch = (
        [pltpu.VMEM((n_chunks, CHUNK), jnp.int32)]
        + [pltpu.VMEM((CHUNK, DIM), jnp.float32) for _ in range(2 * NB)]
        + [pltpu.SemaphoreType.DMA for _ in range(2 * NB)]
    )

    @functools.partial(
        pl.kernel,
        mesh=mesh,
        compiler_params=pltpu.CompilerParams(use_tc_tiling_on_sc=False),
        out_type=jax.ShapeDtypeStruct((n_tokens, DIM), jnp.float32),
        scratch_types=scratch,
    )
    def emb_kernel(idx_hbm, table_hbm, out_hbm, idx_v, *bufs):
        rows_in = bufs[:NB]
        rows_out = bufs[NB:2 * NB]
        in_sem = bufs[2 * NB:3 * NB]
        out_sem = bufs[3 * NB:]

        wid = lax.axis_index("s") * NC + lax.axis_index("c")
        base = wid * per_w
        pltpu.sync_copy(idx_hbm.at[wid], idx_v)

        for b in range(NB):
            pltpu.async_copy(table_hbm.at[idx_v.at[b]], rows_in[b], in_sem[b])

        @pl.loop(0, n_chunks, step=NB)
        def chunk_group(g):
            for b in range(NB):
                j = g + b
                pltpu.make_async_copy(
                    table_hbm.at[idx_v.at[j]], rows_in[b], in_sem[b]
                ).wait()
                @pl.when(j >= NB)
                def _():
                    pltpu.make_async_copy(
                        rows_out[b],
                        out_hbm.at[pl.ds(base, CHUNK)],
                        out_sem[b],
                    ).wait()

                src = rows_in[b]
                dst = rows_out[b]

                @pl.loop(0, CHUNK)
                def row_body(r):
                    for c in range(DIM // LANES):
                        sl = pl.ds(c * LANES, LANES)
                        dst[r, sl] = src[r, sl] * SCALE

                pltpu.async_copy(
                    dst, out_hbm.at[pl.ds(base + j * CHUNK, CHUNK)], out_sem[b]
                )

                @pl.when(j + NB < n_chunks)
                def _():
                    pltpu.async_copy(
                        table_hbm.at[idx_v.at[j + NB]], rows_in[b], in_sem[b]
                    )

        for b in range(NB):
            pltpu.make_async_copy(
                rows_out[b], out_hbm.at[pl.ds(base, CHUNK)], out_sem[b]
            ).wait()

    return emb_kernel


def kernel(tokens, W):
    n_tokens = tokens.shape[0] * tokens.shape[1]
    idx = tokens.reshape(NW, n_tokens // (NW * CHUNK), CHUNK).astype(jnp.int32)
    out = _build(n_tokens, W.shape[0])(idx, W)
    return out.reshape(tokens.shape[0], tokens.shape[1], DIM)


# speedup vs baseline: 1.1656x; 1.0000x over previous
import functools
import math

import jax
import jax.numpy as jnp
from jax import lax
from jax.experimental import pallas as pl
from jax.experimental.pallas import tpu as pltpu
from jax.experimental.pallas import tpu_sc as plsc

DIM = 64
SCALE = math.sqrt(DIM)

NC = 2
NS = 16
NW = NC * NS
CHUNK = 128
LANES = 16
NB = 4


@functools.lru_cache(maxsize=None)
def _build(n_tokens: int, vocab: int):
    per_w = n_tokens // NW
    n_chunks = per_w // CHUNK
    assert per_w * NW == n_tokens and n_chunks * CHUNK == per_w
    assert n_chunks % NB == 0

    mesh = plsc.VectorSubcoreMesh(core_axis_name="c", subcore_axis_name="s")

    scratch = (
        [pltpu.VMEM((n_chunks, CHUNK), jnp.int32)]
        + [pltpu.VMEM((CHUNK, DIM), jnp.float32) for _ in range(2 * NB)]
        + [pltpu.SemaphoreType.DMA for _ in range(2 * NB)]
    )

    @functools.partial(
        pl.kernel,
        mesh=mesh,
        compiler_params=pltpu.CompilerParams(use_tc_tiling_on_sc=False),
        out_type=jax.ShapeDtypeStruct((n_tokens, DIM), jnp.float32),
        scratch_types=scratch,
    )
    def emb_kernel(idx_hbm, table_hbm, out_hbm, idx_v, *bufs):
        rows_in = bufs[:NB]
        rows_out = bufs[NB:2 * NB]
        in_sem = bufs[2 * NB:3 * NB]
        out_sem = bufs[3 * NB:]

        wid = lax.axis_index("s") * NC + lax.axis_index("c")
        base = wid * per_w
        pltpu.sync_copy(idx_hbm.at[wid], idx_v)

        for b in range(NB):
            pltpu.async_copy(table_hbm.at[idx_v.at[b]], rows_in[b], in_sem[b])

        @pl.loop(0, n_chunks, step=NB)
        def chunk_group(g):
            for b in range(NB):
                j = g + b
                pltpu.make_async_copy(
                    table_hbm.at[idx_v.at[j]], rows_in[b], in_sem[b]
                ).wait()
                @pl.when(j >= NB)
                def _():
                    pltpu.make_async_copy(
                        rows_out[b],
                        out_hbm.at[pl.ds(base, CHUNK)],
                        out_sem[b],
                    ).wait()

                src = rows_in[b]
                dst = rows_out[b]

                @plsc.parallel_loop(0, CHUNK, unroll=8)
                def row_body(r):
                    for c in range(DIM // LANES):
                        sl = pl.ds(c * LANES, LANES)
                        dst[r, sl] = src[r, sl] * SCALE

                pltpu.async_copy(
                    dst, out_hbm.at[pl.ds(base + j * CHUNK, CHUNK)], out_sem[b]
                )

                @pl.when(j + NB < n_chunks)
                def _():
                    pltpu.async_copy(
                        table_hbm.at[idx_v.at[j + NB]], rows_in[b], in_sem[b]
                    )

        for b in range(NB):
            pltpu.make_async_copy(
                rows_out[b], out_hbm.at[pl.ds(base, CHUNK)], out_sem[b]
            ).wait()

    return emb_kernel


def kernel(tokens, W):
    n_tokens = tokens.shape[0] * tokens.shape[1]
    idx = tokens.reshape(NW, n_tokens // (NW * CHUNK), CHUNK).astype(jnp.int32)
    out = _build(n_tokens, W.shape[0])(idx, W)
    return out.reshape(tokens.shape[0], tokens.shape[1], DIM)
